# baseline (speedup 1.0000x reference)
import functools

import jax
import jax.numpy as jnp
from jax import lax
from jax.experimental import pallas as pl
from jax.experimental.pallas import tpu as pltpu
from jax.experimental.pallas import tpu_sc as plsc

_TM = 128
_FT = 512
_NW = 32


def _routing_body(ids_ref, dest_ref, deste_ref, desto_ref, offs_ref, sched_ref):
    T = ids_ref.shape[0]
    E = 8
    TM = _TM
    N = T * 2
    nT = N // TM
    S = sched_ref.shape[0]
    ids = ids_ref[...]
    lane_e = lax.broadcasted_iota(jnp.int32, (1, E), 1)
    m0 = (ids[:, 0:1] == lane_e).astype(jnp.int32)
    m1 = (ids[:, 1:2] == lane_e).astype(jnp.int32)
    rowcnt = m0 + m1
    c = rowcnt
    k = 1
    while k < T:
        z = jnp.zeros((k, E), jnp.int32)
        c = c + jnp.concatenate([z, c[:T - k]], axis=0)
        k *= 2
    ex_row = c - rowcnt
    counts = c[T - 1:T, :]
    oc = counts
    k = 1
    while k < E:
        z = jnp.zeros((1, k), jnp.int32)
        oc = oc + jnp.concatenate([z, oc[:, :E - k]], axis=1)
        k *= 2
    offs_ex = oc - counts
    offs_ref[...] = jnp.concatenate(
        [jnp.zeros((1, 1), jnp.int32), oc], axis=1)
    rank0 = ex_row
    rank1 = ex_row + m0
    dest0 = jnp.sum(m0 * (offs_ex + rank0), axis=1, keepdims=True)
    dest1 = jnp.sum(m1 * (offs_ex + rank1), axis=1, keepdims=True)
    dest_ref[...] = jnp.concatenate([dest0, dest1], axis=1)
    deste_ref[...] = dest0
    desto_ref[...] = dest1
    first_tile = offs_ex // TM
    last_tile = jnp.maximum(oc - 1, 0) // TM
    nonempty = counts > 0
    ntiles = jnp.where(nonempty, last_tile - first_tile + 1, 0)
    ce = ntiles
    k = 1
    while k < E:
        z = jnp.zeros((1, k), jnp.int32)
        ce = ce + jnp.concatenate([z, ce[:, :E - k]], axis=1)
        k *= 2
    cum = ce - ntiles
    s_iota = lax.broadcasted_iota(jnp.int32, (S, 1), 0)
    e_s = jnp.sum((s_iota >= ce).astype(jnp.int32), axis=1, keepdims=True)
    e_cl = jnp.minimum(e_s, E - 1)
    sel = (e_cl == lax.broadcasted_iota(jnp.int32, (S, E), 1)).astype(jnp.int32)
    first_sel = jnp.sum(sel * first_tile, axis=1, keepdims=True)
    cum_sel = jnp.sum(sel * cum, axis=1, keepdims=True)
    t_s = first_sel + s_iota - cum_sel
    total = ce[:, E - 1:E]
    valid = s_iota < total
    lane_e8 = lax.broadcasted_iota(jnp.int32, (1, E), 1)
    last_valid_e = jnp.max(jnp.where(nonempty, lane_e8, -1))
    step_e = jnp.where(valid, e_cl, last_valid_e)
    step_t = jnp.where(valid, t_s, nT - 1)
    prev_t = jnp.concatenate(
        [jnp.full((1, 1), -1, jnp.int32), step_t[:S - 1]], axis=0)
    step_init = jnp.where(step_t != prev_t, 1, 0)
    step_init = jnp.where(valid, step_init, 2)
    sched_ref[...] = jnp.concatenate([step_t, step_e, step_init], axis=1)


def _routing(topk_ids, E, TM):
    T, K = topk_ids.shape
    N = T * K
    nT = N // TM
    S = nT + E - 1
    return pl.pallas_call(
        _routing_body,
        out_shape=(
            jax.ShapeDtypeStruct((T, 2), jnp.int32),
            jax.ShapeDtypeStruct((T, 1), jnp.int32),
            jax.ShapeDtypeStruct((T, 1), jnp.int32),
            jax.ShapeDtypeStruct((1, E + 1), jnp.int32),
            jax.ShapeDtypeStruct((S, 3), jnp.int32),
        ),
    )(topk_ids)


@functools.partial(jax.jit, static_argnums=())
def _sc_dispatch(hidden, dest_e, dest_o):
    T, D = hidden.shape
    tok_pw = T // _NW
    mesh = plsc.VectorSubcoreMesh(core_axis_name="c", subcore_axis_name="s")

    @functools.partial(
        pl.kernel, mesh=mesh,
        out_type=jax.ShapeDtypeStruct((2 * T, D), jnp.float32),
        scratch_types=[
            pltpu.VMEM((tok_pw,), jnp.int32),
            pltpu.VMEM((tok_pw,), jnp.int32),
            pltpu.VMEM((tok_pw, D), jnp.float32),
            pltpu.SemaphoreType.DMA,
        ],
    )
    def k(hid_hbm, de_hbm, do_hbm, xs_hbm, ie_v, io_v, rows_v, sem):
        wid = lax.axis_index("s") * 2 + lax.axis_index("c")
        base = wid * tok_pw
        pltpu.sync_copy(de_hbm.at[pl.ds(base, tok_pw)], ie_v)
        pltpu.sync_copy(do_hbm.at[pl.ds(base, tok_pw)], io_v)
        pltpu.sync_copy(hid_hbm.at[pl.ds(base, tok_pw)], rows_v)
        cp1 = pltpu.async_copy(rows_v, xs_hbm.at[ie_v], sem)
        cp2 = pltpu.async_copy(rows_v, xs_hbm.at[io_v], sem)
        cp1.wait()
        cp2.wait()

    return k(hidden, dest_e, dest_o)


def _sc_unsort(ys, dest_flat):
    N, D = ys.shape
    per_w = N // _NW
    chunk = per_w // 2
    mesh = plsc.VectorSubcoreMesh(core_axis_name="c", subcore_axis_name="s")

    @functools.partial(
        pl.kernel, mesh=mesh,
        out_type=jax.ShapeDtypeStruct((N, D), jnp.float32),
        scratch_types=[
            pltpu.VMEM((chunk,), jnp.int32),
            pltpu.VMEM((chunk, D), jnp.float32),
            pltpu.SemaphoreType.DMA,
        ],
    )
    def k(ys_hbm, dest_hbm, out_hbm, idx_v, rows_v, sem):
        wid = lax.axis_index("s") * 2 + lax.axis_index("c")
        for c in range(2):
            base = wid * per_w + c * chunk
            pltpu.sync_copy(dest_hbm.at[pl.ds(base, chunk)], idx_v)
            pltpu.async_copy(ys_hbm.at[idx_v], rows_v, sem).wait()
            pltpu.sync_copy(rows_v, out_hbm.at[pl.ds(base, chunk)])

    return k(ys, dest_flat)


def _grouped_body(sched_ref, offs_ref, xs_ref, w1_ref, w3_ref, w2_ref, ys_ref):
    f = pl.program_id(0)
    s = pl.program_id(1)
    e = sched_ref[s, 1]
    flag = sched_ref[s, 2]
    lo = offs_ref[0, e]
    hi = offs_ref[0, e + 1]
    row0 = sched_ref[s, 0] * _TM
    nonempty = jnp.logical_and(flag != 2,
                               jnp.logical_and(row0 < hi, row0 + _TM > lo))
    init = jnp.logical_and(flag == 1, f == 0)

    @pl.when(nonempty)
    def _():
        x = xs_ref[pl.ds(row0, _TM), :]
        g = lax.dot_general(x, w1_ref[0], (((1,), (1,)), ((), ())),
                            preferred_element_type=jnp.float32)
        u = lax.dot_general(x, w3_ref[0], (((1,), (1,)), ((), ())),
                            preferred_element_type=jnp.float32)
        h = (g * jax.nn.sigmoid(g)) * u
        y = lax.dot_general(h, w2_ref[0], (((1,), (1,)), ((), ())),
                            preferred_element_type=jnp.float32)
        gidx = row0 + lax.broadcasted_iota(jnp.int32, (_TM, 1), 0)
        m = jnp.logical_and(gidx >= lo, gidx < hi).astype(jnp.float32)
        contrib = y * m

        @pl.when(init)
        def _():
            ys_ref[pl.ds(row0, _TM), :] = contrib

        @pl.when(jnp.logical_not(init))
        def _():
            ys_ref[pl.ds(row0, _TM), :] = ys_ref[pl.ds(row0, _TM), :] + contrib


def _grouped_matmul(xs, w1, w3, w2, sched, offs):
    N, D = xs.shape
    E, DFF, _ = w1.shape
    S = sched.shape[0]
    n_f = DFF // _FT
    grid_spec = pltpu.PrefetchScalarGridSpec(
        num_scalar_prefetch=2,
        grid=(n_f, S),
        in_specs=[
            pl.BlockSpec((N, D), lambda f, s, sc, off: (0, 0)),
            pl.BlockSpec((1, _FT, D), lambda f, s, sc, off: (sc[s, 1], f, 0)),
            pl.BlockSpec((1, _FT, D), lambda f, s, sc, off: (sc[s, 1], f, 0)),
            pl.BlockSpec((1, D, _FT), lambda f, s, sc, off: (sc[s, 1], 0, f)),
        ],
        out_specs=pl.BlockSpec((N, D), lambda f, s, sc, off: (0, 0)),
    )
    return pl.pallas_call(
        _grouped_body,
        grid_spec=grid_spec,
        out_shape=jax.ShapeDtypeStruct((N, D), jnp.float32),
        compiler_params=pltpu.CompilerParams(
            dimension_semantics=("arbitrary", "arbitrary"),
        ),
    )(sched, offs, xs, w1, w3, w2)


def _finalize_body(yu_ref, tw_ref, out_ref):
    D = out_ref.shape[1]
    a = yu_ref[:, :D]
    b = yu_ref[:, D:]
    out_ref[...] = a * tw_ref[:, 0:1] + b * tw_ref[:, 1:2]


def _finalize(ys_u2, topk_weights):
    T, D2 = ys_u2.shape
    D = D2 // 2
    TT = T // 2
    return pl.pallas_call(
        _finalize_body,
        grid=(2,),
        in_specs=[
            pl.BlockSpec((TT, D2), lambda i: (i, 0)),
            pl.BlockSpec((TT, 2), lambda i: (i, 0)),
        ],
        out_specs=pl.BlockSpec((TT, D), lambda i: (i, 0)),
        out_shape=jax.ShapeDtypeStruct((T, D), jnp.float32),
    )(ys_u2, topk_weights)


@jax.jit
def kernel(hidden_states, topk_weights, topk_ids, w1, w3, w2):
    T, D = hidden_states.shape
    E = w1.shape[0]
    K = topk_ids.shape[1]
    dest2, dest_e, dest_o, offs, sched = _routing(
        topk_ids.astype(jnp.int32), E, _TM)
    xs = _sc_dispatch(hidden_states, dest_e.reshape(T), dest_o.reshape(T))
    ys = _grouped_matmul(xs, w1, w3, w2, sched, offs)
    ys_u = _sc_unsort(ys, dest2.reshape(T * K))
    ys_u2 = ys_u.reshape(T, K * D)
    return _finalize(ys_u2, topk_weights)


# speedup vs baseline: 1.4707x; 1.4707x over previous
import functools

import jax
import jax.numpy as jnp
from jax import lax
from jax.experimental import pallas as pl
from jax.experimental.pallas import tpu as pltpu
from jax.experimental.pallas import tpu_sc as plsc

_TM = 512
_FT = 512
_NW = 32


def _routing_body(ids_ref, dest_ref, deste_ref, desto_ref, offs_ref, sched_ref):
    T = ids_ref.shape[0]
    E = 8
    TM = _TM
    N = T * 2
    nT = N // TM
    S = sched_ref.shape[0]
    ids = ids_ref[...]
    lane_e = lax.broadcasted_iota(jnp.int32, (1, E), 1)
    m0 = (ids[:, 0:1] == lane_e).astype(jnp.int32)
    m1 = (ids[:, 1:2] == lane_e).astype(jnp.int32)
    rowcnt = m0 + m1
    c = rowcnt
    k = 1
    while k < T:
        z = jnp.zeros((k, E), jnp.int32)
        c = c + jnp.concatenate([z, c[:T - k]], axis=0)
        k *= 2
    ex_row = c - rowcnt
    counts = c[T - 1:T, :]
    oc = counts
    k = 1
    while k < E:
        z = jnp.zeros((1, k), jnp.int32)
        oc = oc + jnp.concatenate([z, oc[:, :E - k]], axis=1)
        k *= 2
    offs_ex = oc - counts
    offs_ref[...] = jnp.concatenate(
        [jnp.zeros((1, 1), jnp.int32), oc], axis=1)
    rank0 = ex_row
    rank1 = ex_row + m0
    dest0 = jnp.sum(m0 * (offs_ex + rank0), axis=1, keepdims=True)
    dest1 = jnp.sum(m1 * (offs_ex + rank1), axis=1, keepdims=True)
    dest_ref[...] = jnp.concatenate([dest0, dest1], axis=1)
    deste_ref[...] = dest0
    desto_ref[...] = dest1
    first_tile = offs_ex // TM
    last_tile = jnp.maximum(oc - 1, 0) // TM
    nonempty = counts > 0
    ntiles = jnp.where(nonempty, last_tile - first_tile + 1, 0)
    ce = ntiles
    k = 1
    while k < E:
        z = jnp.zeros((1, k), jnp.int32)
        ce = ce + jnp.concatenate([z, ce[:, :E - k]], axis=1)
        k *= 2
    cum = ce - ntiles
    s_iota = lax.broadcasted_iota(jnp.int32, (S, 1), 0)
    e_s = jnp.sum((s_iota >= ce).astype(jnp.int32), axis=1, keepdims=True)
    e_cl = jnp.minimum(e_s, E - 1)
    sel = (e_cl == lax.broadcasted_iota(jnp.int32, (S, E), 1)).astype(jnp.int32)
    first_sel = jnp.sum(sel * first_tile, axis=1, keepdims=True)
    cum_sel = jnp.sum(sel * cum, axis=1, keepdims=True)
    t_s = first_sel + s_iota - cum_sel
    total = ce[:, E - 1:E]
    valid = s_iota < total
    lane_e8 = lax.broadcasted_iota(jnp.int32, (1, E), 1)
    last_valid_e = jnp.max(jnp.where(nonempty, lane_e8, -1))
    step_e = jnp.where(valid, e_cl, last_valid_e)
    step_t = jnp.where(valid, t_s, nT - 1)
    prev_t = jnp.concatenate(
        [jnp.full((1, 1), -1, jnp.int32), step_t[:S - 1]], axis=0)
    step_init = jnp.where(step_t != prev_t, 1, 0)
    step_init = jnp.where(valid, step_init, 2)
    sched_ref[...] = jnp.concatenate([step_t, step_e, step_init], axis=1)


def _routing(topk_ids, E, TM):
    T, K = topk_ids.shape
    N = T * K
    nT = N // TM
    S = nT + E - 1
    return pl.pallas_call(
        _routing_body,
        out_shape=(
            jax.ShapeDtypeStruct((T, 2), jnp.int32),
            jax.ShapeDtypeStruct((T, 1), jnp.int32),
            jax.ShapeDtypeStruct((T, 1), jnp.int32),
            jax.ShapeDtypeStruct((1, E + 1), jnp.int32),
            jax.ShapeDtypeStruct((S, 3), jnp.int32),
        ),
    )(topk_ids)


@functools.partial(jax.jit, static_argnums=())
def _sc_dispatch(hidden, dest_e, dest_o):
    T, D = hidden.shape
    tok_pw = T // _NW
    mesh = plsc.VectorSubcoreMesh(core_axis_name="c", subcore_axis_name="s")

    @functools.partial(
        pl.kernel, mesh=mesh,
        out_type=jax.ShapeDtypeStruct((2 * T, D), jnp.float32),
        scratch_types=[
            pltpu.VMEM((tok_pw,), jnp.int32),
            pltpu.VMEM((tok_pw,), jnp.int32),
            pltpu.VMEM((tok_pw, D), jnp.float32),
            pltpu.SemaphoreType.DMA,
        ],
    )
    def k(hid_hbm, de_hbm, do_hbm, xs_hbm, ie_v, io_v, rows_v, sem):
        wid = lax.axis_index("s") * 2 + lax.axis_index("c")
        base = wid * tok_pw
        pltpu.sync_copy(de_hbm.at[pl.ds(base, tok_pw)], ie_v)
        pltpu.sync_copy(do_hbm.at[pl.ds(base, tok_pw)], io_v)
        pltpu.sync_copy(hid_hbm.at[pl.ds(base, tok_pw)], rows_v)
        cp1 = pltpu.async_copy(rows_v, xs_hbm.at[ie_v], sem)
        cp2 = pltpu.async_copy(rows_v, xs_hbm.at[io_v], sem)
        cp1.wait()
        cp2.wait()

    return k(hidden, dest_e, dest_o)


def _sc_unsort(ys, dest_flat):
    N, D = ys.shape
    per_w = N // _NW
    chunk = per_w // 2
    mesh = plsc.VectorSubcoreMesh(core_axis_name="c", subcore_axis_name="s")

    @functools.partial(
        pl.kernel, mesh=mesh,
        out_type=jax.ShapeDtypeStruct((N, D), jnp.float32),
        scratch_types=[
            pltpu.VMEM((chunk,), jnp.int32),
            pltpu.VMEM((chunk, D), jnp.float32),
            pltpu.SemaphoreType.DMA,
        ],
    )
    def k(ys_hbm, dest_hbm, out_hbm, idx_v, rows_v, sem):
        wid = lax.axis_index("s") * 2 + lax.axis_index("c")
        for c in range(2):
            base = wid * per_w + c * chunk
            pltpu.sync_copy(dest_hbm.at[pl.ds(base, chunk)], idx_v)
            pltpu.async_copy(ys_hbm.at[idx_v], rows_v, sem).wait()
            pltpu.sync_copy(rows_v, out_hbm.at[pl.ds(base, chunk)])

    return k(ys, dest_flat)


def _grouped_body(sched_ref, offs_ref, xs_ref, w1_ref, w3_ref, w2_ref, ys_ref):
    f = pl.program_id(0)
    s = pl.program_id(1)
    e = sched_ref[s, 1]
    flag = sched_ref[s, 2]
    lo = offs_ref[0, e]
    hi = offs_ref[0, e + 1]
    row0 = sched_ref[s, 0] * _TM
    nonempty = jnp.logical_and(flag != 2,
                               jnp.logical_and(row0 < hi, row0 + _TM > lo))
    init = jnp.logical_and(flag == 1, f == 0)

    @pl.when(nonempty)
    def _():
        x = xs_ref[pl.ds(row0, _TM), :]
        g = lax.dot_general(x, w1_ref[0], (((1,), (1,)), ((), ())),
                            preferred_element_type=jnp.float32)
        u = lax.dot_general(x, w3_ref[0], (((1,), (1,)), ((), ())),
                            preferred_element_type=jnp.float32)
        h = (g * jax.nn.sigmoid(g)) * u
        y = lax.dot_general(h, w2_ref[0], (((1,), (1,)), ((), ())),
                            preferred_element_type=jnp.float32)
        gidx = row0 + lax.broadcasted_iota(jnp.int32, (_TM, 1), 0)
        m = jnp.logical_and(gidx >= lo, gidx < hi).astype(jnp.float32)
        contrib = y * m

        @pl.when(init)
        def _():
            ys_ref[pl.ds(row0, _TM), :] = contrib

        @pl.when(jnp.logical_not(init))
        def _():
            ys_ref[pl.ds(row0, _TM), :] = ys_ref[pl.ds(row0, _TM), :] + contrib


def _grouped_matmul(xs, w1, w3, w2, sched, offs):
    N, D = xs.shape
    E, DFF, _ = w1.shape
    S = sched.shape[0]
    n_f = DFF // _FT
    grid_spec = pltpu.PrefetchScalarGridSpec(
        num_scalar_prefetch=2,
        grid=(n_f, S),
        in_specs=[
            pl.BlockSpec((N, D), lambda f, s, sc, off: (0, 0)),
            pl.BlockSpec((1, _FT, D), lambda f, s, sc, off: (sc[s, 1], f, 0)),
            pl.BlockSpec((1, _FT, D), lambda f, s, sc, off: (sc[s, 1], f, 0)),
            pl.BlockSpec((1, D, _FT), lambda f, s, sc, off: (sc[s, 1], 0, f)),
        ],
        out_specs=pl.BlockSpec((N, D), lambda f, s, sc, off: (0, 0)),
    )
    return pl.pallas_call(
        _grouped_body,
        grid_spec=grid_spec,
        out_shape=jax.ShapeDtypeStruct((N, D), jnp.float32),
        compiler_params=pltpu.CompilerParams(
            dimension_semantics=("arbitrary", "arbitrary"),
        ),
    )(sched, offs, xs, w1, w3, w2)


def _finalize_body(yu_ref, tw_ref, out_ref):
    D = out_ref.shape[1]
    a = yu_ref[:, :D]
    b = yu_ref[:, D:]
    out_ref[...] = a * tw_ref[:, 0:1] + b * tw_ref[:, 1:2]


def _finalize(ys_u2, topk_weights):
    T, D2 = ys_u2.shape
    D = D2 // 2
    TT = T // 2
    return pl.pallas_call(
        _finalize_body,
        grid=(2,),
        in_specs=[
            pl.BlockSpec((TT, D2), lambda i: (i, 0)),
            pl.BlockSpec((TT, 2), lambda i: (i, 0)),
        ],
        out_specs=pl.BlockSpec((TT, D), lambda i: (i, 0)),
        out_shape=jax.ShapeDtypeStruct((T, D), jnp.float32),
    )(ys_u2, topk_weights)


@jax.jit
def kernel(hidden_states, topk_weights, topk_ids, w1, w3, w2):
    T, D = hidden_states.shape
    E = w1.shape[0]
    K = topk_ids.shape[1]
    dest2, dest_e, dest_o, offs, sched = _routing(
        topk_ids.astype(jnp.int32), E, _TM)
    xs = _sc_dispatch(hidden_states, dest_e.reshape(T), dest_o.reshape(T))
    ys = _grouped_matmul(xs, w1, w3, w2, sched, offs)
    ys_u = _sc_unsort(ys, dest2.reshape(T * K))
    ys_u2 = ys_u.reshape(T, K * D)
    return _finalize(ys_u2, topk_weights)


# speedup vs baseline: 1.6436x; 1.1175x over previous
import functools

import jax
import jax.numpy as jnp
from jax import lax
from jax.experimental import pallas as pl
from jax.experimental.pallas import tpu as pltpu
from jax.experimental.pallas import tpu_sc as plsc

_TM = 512
_FT = 1024
_NW = 32


def _routing_body(ids_ref, dest_ref, deste_ref, desto_ref, offs_ref, sched_ref):
    T = ids_ref.shape[0]
    E = 8
    TM = _TM
    N = T * 2
    nT = N // TM
    S = sched_ref.shape[0]
    ids = ids_ref[...]
    lane_e = lax.broadcasted_iota(jnp.int32, (1, E), 1)
    m0 = (ids[:, 0:1] == lane_e).astype(jnp.int32)
    m1 = (ids[:, 1:2] == lane_e).astype(jnp.int32)
    rowcnt = m0 + m1
    c = rowcnt
    k = 1
    while k < T:
        z = jnp.zeros((k, E), jnp.int32)
        c = c + jnp.concatenate([z, c[:T - k]], axis=0)
        k *= 2
    ex_row = c - rowcnt
    counts = c[T - 1:T, :]
    oc = counts
    k = 1
    while k < E:
        z = jnp.zeros((1, k), jnp.int32)
        oc = oc + jnp.concatenate([z, oc[:, :E - k]], axis=1)
        k *= 2
    offs_ex = oc - counts
    offs_ref[...] = jnp.concatenate(
        [jnp.zeros((1, 1), jnp.int32), oc], axis=1)
    rank0 = ex_row
    rank1 = ex_row + m0
    dest0 = jnp.sum(m0 * (offs_ex + rank0), axis=1, keepdims=True)
    dest1 = jnp.sum(m1 * (offs_ex + rank1), axis=1, keepdims=True)
    dest_ref[...] = jnp.concatenate([dest0, dest1], axis=1)
    deste_ref[...] = dest0
    desto_ref[...] = dest1
    first_tile = offs_ex // TM
    last_tile = jnp.maximum(oc - 1, 0) // TM
    nonempty = counts > 0
    ntiles = jnp.where(nonempty, last_tile - first_tile + 1, 0)
    ce = ntiles
    k = 1
    while k < E:
        z = jnp.zeros((1, k), jnp.int32)
        ce = ce + jnp.concatenate([z, ce[:, :E - k]], axis=1)
        k *= 2
    cum = ce - ntiles
    s_iota = lax.broadcasted_iota(jnp.int32, (S, 1), 0)
    e_s = jnp.sum((s_iota >= ce).astype(jnp.int32), axis=1, keepdims=True)
    e_cl = jnp.minimum(e_s, E - 1)
    sel = (e_cl == lax.broadcasted_iota(jnp.int32, (S, E), 1)).astype(jnp.int32)
    first_sel = jnp.sum(sel * first_tile, axis=1, keepdims=True)
    cum_sel = jnp.sum(sel * cum, axis=1, keepdims=True)
    t_s = first_sel + s_iota - cum_sel
    total = ce[:, E - 1:E]
    valid = s_iota < total
    lane_e8 = lax.broadcasted_iota(jnp.int32, (1, E), 1)
    last_valid_e = jnp.max(jnp.where(nonempty, lane_e8, -1))
    step_e = jnp.where(valid, e_cl, last_valid_e)
    step_t = jnp.where(valid, t_s, nT - 1)
    prev_t = jnp.concatenate(
        [jnp.full((1, 1), -1, jnp.int32), step_t[:S - 1]], axis=0)
    step_init = jnp.where(step_t != prev_t, 1, 0)
    step_init = jnp.where(valid, step_init, 2)
    sched_ref[...] = jnp.concatenate([step_t, step_e, step_init], axis=1)


def _routing(topk_ids, E, TM):
    T, K = topk_ids.shape
    N = T * K
    nT = N // TM
    S = nT + E - 1
    return pl.pallas_call(
        _routing_body,
        out_shape=(
            jax.ShapeDtypeStruct((T, 2), jnp.int32),
            jax.ShapeDtypeStruct((T, 1), jnp.int32),
            jax.ShapeDtypeStruct((T, 1), jnp.int32),
            jax.ShapeDtypeStruct((1, E + 1), jnp.int32),
            jax.ShapeDtypeStruct((S, 3), jnp.int32),
        ),
    )(topk_ids)


@functools.partial(jax.jit, static_argnums=())
def _sc_dispatch(hidden, dest_e, dest_o):
    T, D = hidden.shape
    tok_pw = T // _NW
    mesh = plsc.VectorSubcoreMesh(core_axis_name="c", subcore_axis_name="s")

    @functools.partial(
        pl.kernel, mesh=mesh,
        out_type=jax.ShapeDtypeStruct((2 * T, D), jnp.float32),
        scratch_types=[
            pltpu.VMEM((tok_pw,), jnp.int32),
            pltpu.VMEM((tok_pw,), jnp.int32),
            pltpu.VMEM((tok_pw, D), jnp.float32),
            pltpu.SemaphoreType.DMA,
        ],
    )
    def k(hid_hbm, de_hbm, do_hbm, xs_hbm, ie_v, io_v, rows_v, sem):
        wid = lax.axis_index("s") * 2 + lax.axis_index("c")
        base = wid * tok_pw
        pltpu.sync_copy(de_hbm.at[pl.ds(base, tok_pw)], ie_v)
        pltpu.sync_copy(do_hbm.at[pl.ds(base, tok_pw)], io_v)
        pltpu.sync_copy(hid_hbm.at[pl.ds(base, tok_pw)], rows_v)
        cp1 = pltpu.async_copy(rows_v, xs_hbm.at[ie_v], sem)
        cp2 = pltpu.async_copy(rows_v, xs_hbm.at[io_v], sem)
        cp1.wait()
        cp2.wait()

    return k(hidden, dest_e, dest_o)


def _sc_unsort(ys, dest_flat):
    N, D = ys.shape
    per_w = N // _NW
    chunk = per_w // 2
    mesh = plsc.VectorSubcoreMesh(core_axis_name="c", subcore_axis_name="s")

    @functools.partial(
        pl.kernel, mesh=mesh,
        out_type=jax.ShapeDtypeStruct((N, D), jnp.float32),
        scratch_types=[
            pltpu.VMEM((chunk,), jnp.int32),
            pltpu.VMEM((chunk, D), jnp.float32),
            pltpu.SemaphoreType.DMA,
        ],
    )
    def k(ys_hbm, dest_hbm, out_hbm, idx_v, rows_v, sem):
        wid = lax.axis_index("s") * 2 + lax.axis_index("c")
        for c in range(2):
            base = wid * per_w + c * chunk
            pltpu.sync_copy(dest_hbm.at[pl.ds(base, chunk)], idx_v)
            pltpu.async_copy(ys_hbm.at[idx_v], rows_v, sem).wait()
            pltpu.sync_copy(rows_v, out_hbm.at[pl.ds(base, chunk)])

    return k(ys, dest_flat)


def _grouped_body(sched_ref, offs_ref, xs_ref, w1_ref, w3_ref, w2_ref, ys_ref):
    f = pl.program_id(0)
    s = pl.program_id(1)
    e = sched_ref[s, 1]
    flag = sched_ref[s, 2]
    lo = offs_ref[0, e]
    hi = offs_ref[0, e + 1]
    row0 = sched_ref[s, 0] * _TM
    nonempty = jnp.logical_and(flag != 2,
                               jnp.logical_and(row0 < hi, row0 + _TM > lo))
    init = jnp.logical_and(flag == 1, f == 0)

    @pl.when(nonempty)
    def _():
        x = xs_ref[...]
        g = lax.dot_general(x, w1_ref[0], (((1,), (1,)), ((), ())),
                            preferred_element_type=jnp.float32)
        u = lax.dot_general(x, w3_ref[0], (((1,), (1,)), ((), ())),
                            preferred_element_type=jnp.float32)
        h = (g * jax.nn.sigmoid(g)) * u
        y = lax.dot_general(h, w2_ref[0], (((1,), (1,)), ((), ())),
                            preferred_element_type=jnp.float32)
        gidx = row0 + lax.broadcasted_iota(jnp.int32, (_TM, 1), 0)
        m = jnp.logical_and(gidx >= lo, gidx < hi).astype(jnp.float32)
        contrib = y * m

        @pl.when(init)
        def _():
            ys_ref[pl.ds(row0, _TM), :] = contrib

        @pl.when(jnp.logical_not(init))
        def _():
            ys_ref[pl.ds(row0, _TM), :] = ys_ref[pl.ds(row0, _TM), :] + contrib


def _grouped_matmul(xs, w1, w3, w2, sched, offs):
    N, D = xs.shape
    E, DFF, _ = w1.shape
    S = sched.shape[0]
    n_f = DFF // _FT
    grid_spec = pltpu.PrefetchScalarGridSpec(
        num_scalar_prefetch=2,
        grid=(n_f, S),
        in_specs=[
            pl.BlockSpec((_TM, D), lambda f, s, sc, off: (sc[s, 0], 0)),
            pl.BlockSpec((1, _FT, D), lambda f, s, sc, off: (sc[s, 1], f, 0)),
            pl.BlockSpec((1, _FT, D), lambda f, s, sc, off: (sc[s, 1], f, 0)),
            pl.BlockSpec((1, D, _FT), lambda f, s, sc, off: (sc[s, 1], 0, f)),
        ],
        out_specs=pl.BlockSpec((N, D), lambda f, s, sc, off: (0, 0)),
    )
    return pl.pallas_call(
        _grouped_body,
        grid_spec=grid_spec,
        out_shape=jax.ShapeDtypeStruct((N, D), jnp.float32),
        compiler_params=pltpu.CompilerParams(
            dimension_semantics=("arbitrary", "arbitrary"),
        ),
    )(sched, offs, xs, w1, w3, w2)


def _finalize_body(yu_ref, tw_ref, out_ref):
    D = out_ref.shape[1]
    a = yu_ref[:, :D]
    b = yu_ref[:, D:]
    out_ref[...] = a * tw_ref[:, 0:1] + b * tw_ref[:, 1:2]


def _finalize(ys_u2, topk_weights):
    T, D2 = ys_u2.shape
    D = D2 // 2
    TT = T // 2
    return pl.pallas_call(
        _finalize_body,
        grid=(2,),
        in_specs=[
            pl.BlockSpec((TT, D2), lambda i: (i, 0)),
            pl.BlockSpec((TT, 2), lambda i: (i, 0)),
        ],
        out_specs=pl.BlockSpec((TT, D), lambda i: (i, 0)),
        out_shape=jax.ShapeDtypeStruct((T, D), jnp.float32),
    )(ys_u2, topk_weights)


@jax.jit
def kernel(hidden_states, topk_weights, topk_ids, w1, w3, w2):
    T, D = hidden_states.shape
    E = w1.shape[0]
    K = topk_ids.shape[1]
    dest2, dest_e, dest_o, offs, sched = _routing(
        topk_ids.astype(jnp.int32), E, _TM)
    xs = _sc_dispatch(hidden_states, dest_e.reshape(T), dest_o.reshape(T))
    ys = _grouped_matmul(xs, w1, w3, w2, sched, offs)
    ys_u = _sc_unsort(ys, dest2.reshape(T * K))
    ys_u2 = ys_u.reshape(T, K * D)
    return _finalize(ys_u2, topk_weights)
